# baseline (speedup 1.0000x reference)
import functools

import jax
import jax.numpy as jnp
from jax import lax
from jax.experimental import pallas as pl
from jax.experimental.pallas import tpu as pltpu
from jax.experimental.pallas import tpu_sc as plsc

B = 64
V = 1000000
NC = 2
NS = 16
NW = NC * NS
ROWS_PER_W = B // NW
L = 16

C = 50000
NCHUNK = V // C
G = NCHUNK // 2
U = 5
NV = C // L // U

NEG_INF = float("-inf")


def _process_chunk(buf, base, ys, m0, m1, m2, m3, c_acc):
    yl = ys - base
    in_chunk = (yl >= 0) & (yl < C)
    ylc = jnp.minimum(jnp.maximum(yl, 0), C - 1)
    vbase = (ylc // L) * L
    lane = ylc - vbase
    v = buf[pl.ds(vbase, L)]
    lanesel = lax.iota(jnp.int32, L) == lane
    hit = lanesel & jnp.full((L,), in_chunk)
    c_acc = jnp.where(hit, v, c_acc)
    buf[pl.ds(vbase, L)] = jnp.where(
        hit, jnp.full((L,), NEG_INF, jnp.float32), v)

    def rbody(i, ms):
        a0, a1, a2, a3 = ms
        o = i * (L * U)
        a0 = jnp.maximum(a0, buf[pl.ds(o + 0 * L, L)])
        a1 = jnp.maximum(a1, buf[pl.ds(o + 1 * L, L)])
        a2 = jnp.maximum(a2, buf[pl.ds(o + 2 * L, L)])
        a3 = jnp.maximum(a3, buf[pl.ds(o + 3 * L, L)])
        a0 = jnp.maximum(a0, buf[pl.ds(o + 4 * L, L)])
        return (a0, a1, a2, a3)

    m0, m1, m2, m3 = lax.fori_loop(0, NV, rbody, (m0, m1, m2, m3))
    return m0, m1, m2, m3, c_acc


def _margin_sc(logits, y):
    mesh = plsc.VectorSubcoreMesh(core_axis_name="c", subcore_axis_name="s")

    @functools.partial(
        pl.kernel,
        mesh=mesh,
        out_type=jax.ShapeDtypeStruct((NW, L), jnp.float32),
        compiler_params=pltpu.CompilerParams(needs_layout_passes=False),
        scratch_types=[
            pltpu.VMEM((C,), jnp.float32),
            pltpu.VMEM((C,), jnp.float32),
            pltpu.VMEM((B,), jnp.int32),
            pltpu.VMEM((L,), jnp.float32),
            pltpu.SemaphoreType.DMA,
            pltpu.SemaphoreType.DMA,
        ],
    )
    def k(logits_hbm, y_hbm, out_hbm, buf0, buf1, ybuf, outbuf, sem0, sem1):
        wid = lax.axis_index("s") * NC + lax.axis_index("c")
        pltpu.sync_copy(y_hbm, ybuf)

        losses = []
        for r_i in range(ROWS_PER_W):
            r = wid * ROWS_PER_W + r_i
            vb = (r // L) * L
            yvec = ybuf[pl.ds(vb, L)]
            ys = jnp.max(jnp.where(lax.iota(jnp.int32, L) == (r - vb), yvec,
                                   jnp.full((L,), -1, jnp.int32)))

            pltpu.async_copy(logits_hbm.at[pl.ds(r * V, C)], buf0, sem0)

            def gbody(g, carry, r=r, ys=ys):
                m0, m1, m2, m3, c_acc = carry
                ka = 2 * g
                kb = 2 * g + 1
                pltpu.async_copy(
                    logits_hbm.at[pl.ds(r * V + kb * C, C)], buf1, sem1)
                pltpu.make_async_copy(
                    logits_hbm.at[pl.ds(r * V + ka * C, C)], buf0, sem0).wait()
                m0, m1, m2, m3, c_acc = _process_chunk(
                    buf0, ka * C, ys, m0, m1, m2, m3, c_acc)

                @pl.when(kb + 1 < NCHUNK)
                def _():
                    pltpu.async_copy(
                        logits_hbm.at[pl.ds(r * V + (kb + 1) * C, C)], buf0, sem0)

                pltpu.make_async_copy(
                    logits_hbm.at[pl.ds(r * V + kb * C, C)], buf1, sem1).wait()
                m0, m1, m2, m3, c_acc = _process_chunk(
                    buf1, kb * C, ys, m0, m1, m2, m3, c_acc)
                return (m0, m1, m2, m3, c_acc)

            init = (jnp.full((L,), NEG_INF, jnp.float32),) * 5
            m0, m1, m2, m3, c_acc = lax.fori_loop(0, G, gbody, init)
            m = jnp.maximum(jnp.maximum(m0, m1), jnp.maximum(m2, m3))
            losses.append(jnp.max(c_acc) - jnp.max(m))

        io = lax.iota(jnp.int32, L)
        outv = jnp.zeros((L,), jnp.float32)
        for r_i, lv in enumerate(losses):
            outv = jnp.where(io == r_i, jnp.full((L,), lv, jnp.float32), outv)
        outbuf[...] = outv
        pltpu.sync_copy(outbuf, out_hbm.at[wid])

    return k(logits, y)


def kernel(logits, y):
    out = _margin_sc(logits.reshape(-1), y.astype(jnp.int32))
    return out[:, :ROWS_PER_W].reshape(B)
